# Initial kernel scaffold; baseline (speedup 1.0000x reference)
import jax
import jax.numpy as jnp
from jax.experimental import pallas as pl


def kernel(output, target):
    raise NotImplementedError("write your pallas kernel here")


# speedup vs baseline: 5.1220x; 5.1220x over previous
import functools

import jax
import jax.numpy as jnp
from jax.experimental import pallas as pl

B, S, C, P = 64, 32, 8192, 8
N = B * S
R = 64
GRID = N // R
NEG_INF = float("-inf")


def _fused_body(x_ref, t_ref, loss_ref, corr_ref, preds_ref, counts_ref):
    i = pl.program_id(0)
    x = x_ref[...]
    t = t_ref[...]
    iota = jax.lax.broadcasted_iota(jnp.int32, (R, C), 1)

    rmax = jnp.max(x, axis=1, keepdims=True)
    sumexp = jnp.sum(jnp.exp(x - rmax), axis=1, keepdims=True)
    lse = jnp.log(sumexp) + rmax

    tsum = jnp.zeros((R, 1), jnp.float32)
    for j in range(P):
        tj = t[:, j:j + 1]
        tsum = tsum + jnp.sum(jnp.where(iota == tj, x, 0.0), axis=1,
                              keepdims=True)
    loss_blk = jnp.sum(lse - tsum / jnp.float32(P))

    xw = x
    idxs = []
    for _ in range(P):
        m = jnp.max(xw, axis=1, keepdims=True)
        cand = jnp.where(xw == m, iota, C)
        idx = jnp.min(cand, axis=1, keepdims=True)
        idxs.append(idx)
        xw = jnp.where(cand == idx, NEG_INF, xw)
    preds_ref[...] = jnp.concatenate(idxs, axis=1)

    hit = (xw == NEG_INF)
    cnt_blk = jnp.sum(jnp.where(hit, 1.0, 0.0), axis=0, keepdims=True)

    corr_blk = jnp.float32(0.0)
    for k in range(P):
        mk = jnp.zeros((R, 1), jnp.bool_)
        for j in range(P):
            mk = mk | (idxs[k] == t[:, j:j + 1])
        corr_blk = corr_blk + jnp.sum(jnp.where(mk, 1.0, 0.0))

    @pl.when(i == 0)
    def _():
        loss_ref[...] = jnp.zeros_like(loss_ref)
        corr_ref[...] = jnp.zeros_like(corr_ref)
        counts_ref[...] = jnp.zeros_like(counts_ref)

    loss_ref[...] += loss_blk
    corr_ref[...] += corr_blk
    counts_ref[...] += cnt_blk


@functools.partial(jax.jit, static_argnames=("interpret",))
def _run(x2, t2, interpret=False):
    loss_sum, correct, preds, counts = pl.pallas_call(
        _fused_body,
        grid=(GRID,),
        in_specs=[
            pl.BlockSpec((R, C), lambda i: (i, 0)),
            pl.BlockSpec((R, P), lambda i: (i, 0)),
        ],
        out_specs=[
            pl.BlockSpec((1, 1), lambda i: (0, 0)),
            pl.BlockSpec((1, 1), lambda i: (0, 0)),
            pl.BlockSpec((R, P), lambda i: (i, 0)),
            pl.BlockSpec((1, C), lambda i: (0, 0)),
        ],
        out_shape=[
            jax.ShapeDtypeStruct((1, 1), jnp.float32),
            jax.ShapeDtypeStruct((1, 1), jnp.float32),
            jax.ShapeDtypeStruct((N, P), jnp.int32),
            jax.ShapeDtypeStruct((1, C), jnp.float32),
        ],
        interpret=interpret,
    )(x2, t2)
    return loss_sum, correct, preds, counts


def kernel(output, target, interpret=False):
    bb, ss, cc = output.shape
    x2 = output.reshape(N, C)
    t2 = target.reshape(N, P)
    loss_sum, correct, preds, counts = _run(x2, t2, interpret=interpret)
    loss = (loss_sum[0, 0] / jnp.float32(N))
    acc = correct[0, 0] / jnp.float32(N * P) * 100.0
    cvec = counts[0]
    p_counts = (cvec / cvec.sum() * 100.0).astype(jnp.int32)
    prompt_id_preds = preds.reshape(bb, ss, P)
    return (loss, prompt_id_preds, acc, p_counts)
